# baseline (speedup 1.0000x reference)
import functools

import jax
import jax.numpy as jnp
from jax import lax
from jax.experimental import pallas as pl
from jax.experimental.pallas import tpu as pltpu
from jax.experimental.pallas import tpu_sc as plsc

_NW = 32
_CHUNK = 128


def _sc_gather(table, ids_flat):
    ntok = ids_flat.shape[0]
    wd = table.shape[1]
    dt = table.dtype
    tok_per_w = ntok // _NW
    n_chunk = tok_per_w // _CHUNK
    mesh = plsc.VectorSubcoreMesh(core_axis_name="c", subcore_axis_name="s")

    nb = 4

    @functools.partial(
        pl.kernel,
        mesh=mesh,
        out_type=jax.ShapeDtypeStruct((ntok, wd), dt),
        scratch_types=[
            pltpu.VMEM((n_chunk, _CHUNK), jnp.int32),
            pltpu.VMEM((nb, _CHUNK, wd), dt),
            pltpu.SemaphoreType.DMA,
            pltpu.SemaphoreType.DMA,
        ],
    )
    def k(table_hbm, idx_hbm, out_hbm, idx_v, rows_v, gsem, osem):
        wid = lax.axis_index("s") * 2 + lax.axis_index("c")
        base = wid * tok_per_w

        pltpu.sync_copy(idx_hbm.at[pl.ds(wid * n_chunk, n_chunk)], idx_v)

        def gath(g, slot):
            pltpu.async_copy(table_hbm.at[idx_v.at[g]], rows_v.at[slot], gsem)

        def gath_wait(g, slot):
            pltpu.make_async_copy(table_hbm.at[idx_v.at[g]],
                                  rows_v.at[slot], gsem).wait()

        def wr(g, slot):
            pltpu.async_copy(rows_v.at[slot],
                             out_hbm.at[pl.ds(base + g * _CHUNK, _CHUNK)], osem)

        def wr_wait(g, slot):
            pltpu.make_async_copy(
                rows_v.at[slot],
                out_hbm.at[pl.ds(base + g * _CHUNK, _CHUNK)], osem).wait()

        for p in range(nb - 1):
            gath(p, p)

        def body(gg, _):
            for b in range(nb):
                g = gg * nb + b
                gath_wait(g, b)
                wr(g, b)
                @pl.when(g > 0)
                def _():
                    wr_wait(g - 1, (b + nb - 1) % nb)

                @pl.when(g + nb - 1 < n_chunk)
                def _():
                    gath(g + nb - 1, (b + nb - 1) % nb)
            return 0

        lax.fori_loop(0, n_chunk // nb, body, 0)
        wr_wait(n_chunk - 1, nb - 1)

    return k(table, ids_flat.reshape(ntok // _CHUNK, _CHUNK))


def _dense_body(g_ref, oh_ref, w_ref, pt_ref, gm_ref, bt_ref, o_ref):
    bb, L, dim = o_ref.shape
    g = g_ref[...]
    oh = oh_ref[...].astype(jnp.float32)
    x = jnp.dot(g, w_ref[...], preferred_element_type=jnp.float32)
    x = x + jnp.dot(oh, pt_ref[...], preferred_element_type=jnp.float32)
    mean = jnp.sum(x, axis=-1, keepdims=True) * (1.0 / dim)
    xc = x - mean
    var = jnp.sum(xc * xc, axis=-1, keepdims=True) * (1.0 / dim)
    y = xc * lax.rsqrt(var + 1e-12)
    y = y * gm_ref[...] + bt_ref[...]
    o_ref[...] = y.reshape(bb, L, dim)


def _tc_dense(g2, oh2, W2, PTa, gamma2, beta2, batch, L):
    ntok, wd = g2.shape
    dim = W2.shape[1]
    bb = 128
    grid = (batch // bb,)
    return pl.pallas_call(
        _dense_body,
        grid=grid,
        in_specs=[
            pl.BlockSpec((bb * L, wd), lambda i: (i, 0)),
            pl.BlockSpec((bb * L, 128), lambda i: (i, 0)),
            pl.BlockSpec((wd, dim), lambda i: (0, 0)),
            pl.BlockSpec((128, dim), lambda i: (0, 0)),
            pl.BlockSpec((1, dim), lambda i: (0, 0)),
            pl.BlockSpec((1, dim), lambda i: (0, 0)),
        ],
        out_specs=pl.BlockSpec((bb, L, dim), lambda i: (i, 0, 0)),
        out_shape=jax.ShapeDtypeStruct((batch, L, dim), jnp.float32),
    )(g2, oh2, W2, PTa, gamma2, beta2)


def kernel(input_ids, segment_ids, word_emb, W2, pos_emb, type_emb, gamma, beta):
    batch, L = input_ids.shape
    dim = W2.shape[1]
    ids_flat = input_ids.reshape(-1).astype(jnp.int32)
    g2 = _sc_gather(word_emb, ids_flat)

    ptid = jnp.arange(L, dtype=jnp.int32)[None, :] * 3 + segment_ids.astype(jnp.int32)
    oh2 = jax.nn.one_hot(ptid.reshape(-1), 128, dtype=jnp.int8)
    PTa = jnp.zeros((128, dim), jnp.float32)
    PTa = PTa.at[: 3 * L].set(
        (pos_emb[:, None, :] + type_emb[None, :, :]).reshape(3 * L, dim))

    return _tc_dense(g2, oh2, W2, PTa,
                     gamma.reshape(1, -1), beta.reshape(1, -1), batch, L)


# speedup vs baseline: 1.0454x; 1.0454x over previous
import functools

import jax
import jax.numpy as jnp
from jax import lax
from jax.experimental import pallas as pl
from jax.experimental.pallas import tpu as pltpu
from jax.experimental.pallas import tpu_sc as plsc

_NW = 32
_CHUNK = 128


def _sc_gather(table, ids_flat, out_wd=None):
    ntok = ids_flat.shape[0]
    wd = table.shape[1]
    out_wd = wd if out_wd is None else out_wd
    dt = table.dtype
    tok_per_w = ntok // _NW
    n_chunk = tok_per_w // _CHUNK
    mesh = plsc.VectorSubcoreMesh(core_axis_name="c", subcore_axis_name="s")

    nb = 4

    @functools.partial(
        pl.kernel,
        mesh=mesh,
        out_type=jax.ShapeDtypeStruct((ntok, out_wd), dt),
        scratch_types=[
            pltpu.VMEM((n_chunk, _CHUNK), jnp.int32),
            pltpu.VMEM((nb, _CHUNK, wd), dt),
            pltpu.SemaphoreType.DMA,
            pltpu.SemaphoreType.DMA,
        ],
    )
    def k(table_hbm, idx_hbm, out_hbm, idx_v, rows_v, gsem, osem):
        wid = lax.axis_index("s") * 2 + lax.axis_index("c")
        base = wid * tok_per_w

        pltpu.sync_copy(idx_hbm.at[pl.ds(wid * n_chunk, n_chunk)], idx_v)

        def gath(g, slot):
            pltpu.async_copy(table_hbm.at[idx_v.at[g]], rows_v.at[slot], gsem)

        def gath_wait(g, slot):
            pltpu.make_async_copy(table_hbm.at[idx_v.at[g]],
                                  rows_v.at[slot], gsem).wait()

        owd = out_hbm.shape[1]

        def wr(g, slot):
            pltpu.async_copy(rows_v.at[slot, :, pl.ds(0, owd)],
                             out_hbm.at[pl.ds(base + g * _CHUNK, _CHUNK)], osem)

        def wr_wait(g, slot):
            pltpu.make_async_copy(
                rows_v.at[slot, :, pl.ds(0, owd)],
                out_hbm.at[pl.ds(base + g * _CHUNK, _CHUNK)], osem).wait()

        for p in range(nb - 1):
            gath(p, p)

        def body(gg, _):
            for b in range(nb):
                g = gg * nb + b
                gath_wait(g, b)
                wr(g, b)
                @pl.when(g > 0)
                def _():
                    wr_wait(g - 1, (b + nb - 1) % nb)

                @pl.when(g + nb - 1 < n_chunk)
                def _():
                    gath(g + nb - 1, (b + nb - 1) % nb)
            return 0

        lax.fori_loop(0, n_chunk // nb, body, 0)
        wr_wait(n_chunk - 1, nb - 1)

    return k(table, ids_flat.reshape(ntok // _CHUNK, _CHUNK))


def _dense_body(g_ref, s_ref, w_ref, pt_ref, gm_ref, bt_ref, o_ref):
    bb, L, dim = o_ref.shape
    g = g_ref[...]
    s = s_ref[...]
    ptid = lax.broadcasted_iota(jnp.int32, (bb, L), 1) * 3 + s
    pt3 = lax.broadcast_in_dim(ptid, (bb, L, 128), (0, 1))
    cls = lax.broadcasted_iota(jnp.int32, (bb, L, 128), 2)
    oh = jnp.where(pt3 == cls, 1.0, 0.0).reshape(bb * L, 128)
    x = jnp.dot(g, w_ref[...], preferred_element_type=jnp.float32)
    x = x + jnp.dot(oh, pt_ref[...], preferred_element_type=jnp.float32)
    mean = jnp.sum(x, axis=-1, keepdims=True) * (1.0 / dim)
    xc = x - mean
    var = jnp.sum(xc * xc, axis=-1, keepdims=True) * (1.0 / dim)
    y = xc * lax.rsqrt(var + 1e-12)
    y = y * gm_ref[...] + bt_ref[...]
    o_ref[...] = y.reshape(bb, L, dim)


def _tc_dense(g2, seg, W2, PTa, gamma2, beta2, batch, L):
    ntok, wd = g2.shape
    dim = W2.shape[1]
    bb = 256
    grid = (batch // bb,)
    return pl.pallas_call(
        _dense_body,
        grid=grid,
        in_specs=[
            pl.BlockSpec((bb * L, wd), lambda i: (i, 0)),
            pl.BlockSpec((bb, L), lambda i: (i, 0)),
            pl.BlockSpec((wd, dim), lambda i: (0, 0)),
            pl.BlockSpec((128, dim), lambda i: (0, 0)),
            pl.BlockSpec((1, dim), lambda i: (0, 0)),
            pl.BlockSpec((1, dim), lambda i: (0, 0)),
        ],
        out_specs=pl.BlockSpec((bb, L, dim), lambda i: (i, 0, 0)),
        out_shape=jax.ShapeDtypeStruct((batch, L, dim), jnp.float32),
    )(g2, seg, W2, PTa, gamma2, beta2)


def kernel(input_ids, segment_ids, word_emb, W2, pos_emb, type_emb, gamma, beta):
    batch, L = input_ids.shape
    dim = W2.shape[1]
    ids_flat = input_ids.reshape(-1).astype(jnp.int32)
    g2 = _sc_gather(word_emb, ids_flat)

    PTa = jnp.zeros((128, dim), jnp.float32)
    PTa = PTa.at[: 3 * L].set(
        (pos_emb[:, None, :] + type_emb[None, :, :]).reshape(3 * L, dim))

    return _tc_dense(g2, segment_ids.astype(jnp.int32), W2, PTa,
                     gamma.reshape(1, -1), beta.reshape(1, -1), batch, L)


# speedup vs baseline: 1.0513x; 1.0056x over previous
import functools

import jax
import jax.numpy as jnp
from jax import lax
from jax.experimental import pallas as pl
from jax.experimental.pallas import tpu as pltpu
from jax.experimental.pallas import tpu_sc as plsc

_NW = 32
_CHUNK = 128


def _sc_gather(table, ids_flat, out_wd=None):
    ntok = ids_flat.shape[0]
    wd = table.shape[1]
    out_wd = wd if out_wd is None else out_wd
    dt = table.dtype
    tok_per_w = ntok // _NW
    n_chunk = tok_per_w // _CHUNK
    mesh = plsc.VectorSubcoreMesh(core_axis_name="c", subcore_axis_name="s")

    nb = 4

    @functools.partial(
        pl.kernel,
        mesh=mesh,
        out_type=jax.ShapeDtypeStruct((ntok, out_wd), dt),
        scratch_types=[
            pltpu.VMEM((n_chunk, _CHUNK), jnp.int32),
            pltpu.VMEM((nb, _CHUNK, wd), dt),
            pltpu.SemaphoreType.DMA,
            pltpu.SemaphoreType.DMA,
        ],
    )
    def k(table_hbm, idx_hbm, out_hbm, idx_v, rows_v, gsem, osem):
        wid = lax.axis_index("s") * 2 + lax.axis_index("c")
        base = wid * tok_per_w

        pltpu.sync_copy(idx_hbm.at[pl.ds(wid * n_chunk, n_chunk)], idx_v)

        def gath(g, slot):
            pltpu.async_copy(table_hbm.at[idx_v.at[g]], rows_v.at[slot], gsem)

        def gath_wait(g, slot):
            pltpu.make_async_copy(table_hbm.at[idx_v.at[g]],
                                  rows_v.at[slot], gsem).wait()

        owd = out_hbm.shape[1]

        def wr(g, slot):
            pltpu.async_copy(rows_v.at[slot, :, pl.ds(0, owd)],
                             out_hbm.at[pl.ds(base + g * _CHUNK, _CHUNK)], osem)

        def wr_wait(g, slot):
            pltpu.make_async_copy(
                rows_v.at[slot, :, pl.ds(0, owd)],
                out_hbm.at[pl.ds(base + g * _CHUNK, _CHUNK)], osem).wait()

        for p in range(nb - 1):
            gath(p, p)

        def body(gg, _):
            for b in range(nb):
                g = gg * nb + b
                gath_wait(g, b)
                wr(g, b)
                @pl.when(g > 0)
                def _():
                    wr_wait(g - 1, (b + nb - 1) % nb)

                @pl.when(g + nb - 1 < n_chunk)
                def _():
                    gath(g + nb - 1, (b + nb - 1) % nb)
            return 0

        lax.fori_loop(0, n_chunk // nb, body, 0)
        wr_wait(n_chunk - 1, nb - 1)

    return k(table, ids_flat.reshape(ntok // _CHUNK, _CHUNK))


def _dense_body(g_ref, s_ref, w_ref, pt_ref, gm_ref, bt_ref, o_ref):
    bb, L, dim = o_ref.shape
    g = g_ref[...]
    s = s_ref[...]
    ptid = lax.broadcasted_iota(jnp.int32, (bb, L), 1) * 3 + s
    pt3 = lax.broadcast_in_dim(ptid, (bb, L, 128), (0, 1))
    cls = lax.broadcasted_iota(jnp.int32, (bb, L, 128), 2)
    oh = jnp.where(pt3 == cls, 1.0, 0.0).reshape(bb * L, 128)
    x = jnp.dot(g, w_ref[...], preferred_element_type=jnp.float32)
    x = x + jnp.dot(oh, pt_ref[...], preferred_element_type=jnp.float32)
    mean = jnp.sum(x, axis=-1, keepdims=True) * (1.0 / dim)
    xc = x - mean
    var = jnp.sum(xc * xc, axis=-1, keepdims=True) * (1.0 / dim)
    y = xc * lax.rsqrt(var + 1e-12)
    y = y * gm_ref[...] + bt_ref[...]
    o_ref[...] = y.reshape(bb, L, dim)


def _dense_body_acc(prev_ref, g_ref, s_ref, w_ref, pt_ref, gm_ref, bt_ref, o_ref):
    del prev_ref
    _dense_body(g_ref, s_ref, w_ref, pt_ref, gm_ref, bt_ref, o_ref)


def _tc_dense(g2, seg, W2, PTa, gamma2, beta2, batch, L, blk_off=0, prev=None):
    ntok, wd = g2.shape
    dim = W2.shape[1]
    bb = 256
    nblk = seg.shape[0] // bb
    grid = (nblk,)
    in_specs = [
        pl.BlockSpec((bb * L, wd), lambda i: (i, 0)),
        pl.BlockSpec((bb, L), lambda i: (i, 0)),
        pl.BlockSpec((wd, dim), lambda i: (0, 0)),
        pl.BlockSpec((128, dim), lambda i: (0, 0)),
        pl.BlockSpec((1, dim), lambda i: (0, 0)),
        pl.BlockSpec((1, dim), lambda i: (0, 0)),
    ]
    args = (g2, seg, W2, PTa, gamma2, beta2)
    body = _dense_body
    kwargs = {}
    if prev is not None:
        in_specs = [pl.BlockSpec(memory_space=pl.ANY)] + in_specs
        args = (prev,) + args
        body = _dense_body_acc
        kwargs["input_output_aliases"] = {0: 0}
    return pl.pallas_call(
        body,
        grid=grid,
        in_specs=in_specs,
        out_specs=pl.BlockSpec((bb, L, dim), lambda i: (i + blk_off, 0, 0)),
        out_shape=jax.ShapeDtypeStruct((batch, L, dim), jnp.float32),
        **kwargs,
    )(*args)


def kernel(input_ids, segment_ids, word_emb, W2, pos_emb, type_emb, gamma, beta):
    batch, L = input_ids.shape
    dim = W2.shape[1]
    ids_flat = input_ids.reshape(-1).astype(jnp.int32)
    seg = segment_ids.astype(jnp.int32)
    gamma2 = gamma.reshape(1, -1)
    beta2 = beta.reshape(1, -1)

    PTa = jnp.zeros((128, dim), jnp.float32)
    PTa = PTa.at[: 3 * L].set(
        (pos_emb[:, None, :] + type_emb[None, :, :]).reshape(3 * L, dim))

    hb = batch // 2
    ht = hb * L
    g_a = _sc_gather(word_emb, ids_flat[:ht])
    g_b = _sc_gather(word_emb, ids_flat[ht:])
    out = _tc_dense(g_a, seg[:hb], W2, PTa, gamma2, beta2, batch, L)
    out = _tc_dense(g_b, seg[hb:], W2, PTa, gamma2, beta2, batch, L,
                    blk_off=hb // 256, prev=out)
    return out


# speedup vs baseline: 2.4427x; 2.3236x over previous
import functools

import jax
import jax.numpy as jnp
from jax import lax
from jax.experimental import pallas as pl
from jax.experimental.pallas import tpu as pltpu
from jax.experimental.pallas import tpu_sc as plsc

_NW = 32
_CHUNK = 128


def _sc_gather(table, ids_flat, out_wd=None):
    ntok = ids_flat.shape[0]
    wd = table.shape[1]
    out_wd = wd if out_wd is None else out_wd
    dt = table.dtype
    tok_per_w = ntok // _NW
    n_chunk = tok_per_w // _CHUNK
    mesh = plsc.VectorSubcoreMesh(core_axis_name="c", subcore_axis_name="s")

    nb = 4

    @functools.partial(
        pl.kernel,
        mesh=mesh,
        out_type=jax.ShapeDtypeStruct((ntok, out_wd), dt),
        scratch_types=[
            pltpu.VMEM((n_chunk, _CHUNK), jnp.int32),
            pltpu.VMEM((nb, _CHUNK, wd), dt),
            pltpu.SemaphoreType.DMA,
            pltpu.SemaphoreType.DMA,
        ],
    )
    def k(table_hbm, idx_hbm, out_hbm, idx_v, rows_v, gsem, osem):
        wid = lax.axis_index("s") * 2 + lax.axis_index("c")
        base = wid * tok_per_w

        pltpu.sync_copy(idx_hbm.at[pl.ds(wid * n_chunk, n_chunk)], idx_v)

        def gath(g, slot):
            pltpu.async_copy(table_hbm.at[idx_v.at[g]], rows_v.at[slot], gsem)

        def gath_wait(g, slot):
            pltpu.make_async_copy(table_hbm.at[idx_v.at[g]],
                                  rows_v.at[slot], gsem).wait()

        owd = out_hbm.shape[1]

        def wr(g, slot):
            pltpu.async_copy(rows_v.at[slot, :, pl.ds(0, owd)],
                             out_hbm.at[pl.ds(base + g * _CHUNK, _CHUNK)], osem)

        def wr_wait(g, slot):
            pltpu.make_async_copy(
                rows_v.at[slot, :, pl.ds(0, owd)],
                out_hbm.at[pl.ds(base + g * _CHUNK, _CHUNK)], osem).wait()

        for p in range(nb - 1):
            gath(p, p)

        def body(gg, _):
            for b in range(nb):
                g = gg * nb + b
                gath_wait(g, b)
                wr(g, b)
                @pl.when(g > 0)
                def _():
                    wr_wait(g - 1, (b + nb - 1) % nb)

                @pl.when(g + nb - 1 < n_chunk)
                def _():
                    gath(g + nb - 1, (b + nb - 1) % nb)
            return 0

        lax.fori_loop(0, n_chunk // nb, body, 0)
        wr_wait(n_chunk - 1, nb - 1)

    return k(table, ids_flat.reshape(ntok // _CHUNK, _CHUNK))


_BT = 4096


def _dense_body(g_ref, s_ref, w_ref, pt_ref, gm_ref, bt_ref, o_ref):
    _, dim, bt = o_ref.shape
    g = g_ref[...]
    xt = lax.dot_general(w_ref[...], g, (((0,), (1,)), ((), ())),
                         preferred_element_type=jnp.float32)
    l = pl.program_id(0)
    s = s_ref[...].reshape(1, bt)
    ptid = lax.broadcast_in_dim(l * 3 + s, (128, bt), (0, 1))
    cls = lax.broadcasted_iota(jnp.int32, (128, bt), 0)
    oht = jnp.where(ptid == cls, 1.0, 0.0)
    xt = xt + lax.dot_general(pt_ref[...], oht, (((0,), (0,)), ((), ())),
                              preferred_element_type=jnp.float32)
    mean = jnp.sum(xt, axis=0, keepdims=True) * (1.0 / dim)
    xc = xt - mean
    var = jnp.sum(xc * xc, axis=0, keepdims=True) * (1.0 / dim)
    y = xc * lax.rsqrt(var + 1e-12)
    y = y * gm_ref[...] + bt_ref[...]
    o_ref[...] = y.reshape(1, dim, bt)


def _dense_body_acc(prev_ref, g_ref, s_ref, w_ref, pt_ref, gm_ref, bt_ref, o_ref):
    del prev_ref
    _dense_body(g_ref, s_ref, w_ref, pt_ref, gm_ref, bt_ref, o_ref)


def _tc_dense(g2, seg3, W2, PTa, gammaT, betaT, batch, L, b_off=0, prev=None):
    wd = W2.shape[0]
    dim = W2.shape[1]
    hb = seg3.shape[2]
    nbb = hb // _BT
    grid = (L, nbb)
    in_specs = [
        pl.BlockSpec((_BT, wd), lambda l, j: (l * nbb + j, 0)),
        pl.BlockSpec((1, 1, _BT), lambda l, j: (l, 0, j)),
        pl.BlockSpec((wd, dim), lambda l, j: (0, 0)),
        pl.BlockSpec((128, dim), lambda l, j: (0, 0)),
        pl.BlockSpec((dim, 1), lambda l, j: (0, 0)),
        pl.BlockSpec((dim, 1), lambda l, j: (0, 0)),
    ]
    args = (g2, seg3, W2, PTa, gammaT, betaT)
    body = _dense_body
    kwargs = {}
    if prev is not None:
        in_specs = [pl.BlockSpec(memory_space=pl.ANY)] + in_specs
        args = (prev,) + args
        body = _dense_body_acc
        kwargs["input_output_aliases"] = {0: 0}
    return pl.pallas_call(
        body,
        grid=grid,
        in_specs=in_specs,
        out_specs=pl.BlockSpec((1, dim, _BT), lambda l, j: (l, 0, j + b_off)),
        out_shape=jax.ShapeDtypeStruct((L, dim, batch), jnp.float32),
        **kwargs,
    )(*args)


def kernel(input_ids, segment_ids, word_emb, W2, pos_emb, type_emb, gamma, beta):
    batch, L = input_ids.shape
    dim = W2.shape[1]
    idsT = input_ids.astype(jnp.int32).T
    segT = segment_ids.astype(jnp.int32).T
    gammaT = gamma.reshape(-1, 1)
    betaT = beta.reshape(-1, 1)

    PTa = jnp.zeros((128, dim), jnp.float32)
    PTa = PTa.at[: 3 * L].set(
        (pos_emb[:, None, :] + type_emb[None, :, :]).reshape(3 * L, dim))

    hb = batch // 2
    g_a = _sc_gather(word_emb, idsT[:, :hb].reshape(-1))
    g_b = _sc_gather(word_emb, idsT[:, hb:].reshape(-1))
    seg3_a = segT[:, :hb].reshape(L, 1, hb)
    seg3_b = segT[:, hb:].reshape(L, 1, hb)
    out = _tc_dense(g_a, seg3_a, W2, PTa, gammaT, betaT, batch, L)
    out = _tc_dense(g_b, seg3_b, W2, PTa, gammaT, betaT, batch, L,
                    b_off=hb // _BT, prev=out)
    return jnp.transpose(out, (2, 0, 1))


# speedup vs baseline: 2.4567x; 1.0057x over previous
import functools

import jax
import jax.numpy as jnp
from jax import lax
from jax.experimental import pallas as pl
from jax.experimental.pallas import tpu as pltpu
from jax.experimental.pallas import tpu_sc as plsc

_NW = 32
_CHUNK = 128


def _sc_gather(table, ids_flat, out_wd=None):
    ntok = ids_flat.shape[0]
    wd = table.shape[1]
    out_wd = wd if out_wd is None else out_wd
    dt = table.dtype
    tok_per_w = ntok // _NW
    n_chunk = tok_per_w // _CHUNK
    mesh = plsc.VectorSubcoreMesh(core_axis_name="c", subcore_axis_name="s")

    nb = 4

    @functools.partial(
        pl.kernel,
        mesh=mesh,
        out_type=jax.ShapeDtypeStruct((ntok, out_wd), dt),
        scratch_types=[
            pltpu.VMEM((n_chunk, _CHUNK), jnp.int32),
            pltpu.VMEM((nb, _CHUNK, wd), dt),
            pltpu.SemaphoreType.DMA,
            pltpu.SemaphoreType.DMA,
        ],
    )
    def k(table_hbm, idx_hbm, out_hbm, idx_v, rows_v, gsem, osem):
        wid = lax.axis_index("s") * 2 + lax.axis_index("c")
        base = wid * tok_per_w

        pltpu.sync_copy(idx_hbm.at[wid], idx_v)

        def gath(g, slot):
            pltpu.async_copy(table_hbm.at[idx_v.at[g]], rows_v.at[slot], gsem)

        def gath_wait(g, slot):
            pltpu.make_async_copy(table_hbm.at[idx_v.at[g]],
                                  rows_v.at[slot], gsem).wait()

        owd = out_hbm.shape[1]

        def wr(g, slot):
            pltpu.async_copy(rows_v.at[slot, :, pl.ds(0, owd)],
                             out_hbm.at[pl.ds(base + g * _CHUNK, _CHUNK)], osem)

        def wr_wait(g, slot):
            pltpu.make_async_copy(
                rows_v.at[slot, :, pl.ds(0, owd)],
                out_hbm.at[pl.ds(base + g * _CHUNK, _CHUNK)], osem).wait()

        for p in range(nb - 1):
            gath(p, p)

        def body(gg, _):
            for b in range(nb):
                g = gg * nb + b
                gath_wait(g, b)
                wr(g, b)
                @pl.when(g > 0)
                def _():
                    wr_wait(g - 1, (b + nb - 1) % nb)

                @pl.when(g + nb - 1 < n_chunk)
                def _():
                    gath(g + nb - 1, (b + nb - 1) % nb)
            return 0

        lax.fori_loop(0, n_chunk // nb, body, 0)
        wr_wait(n_chunk - 1, nb - 1)

    return k(table, ids_flat.reshape(_NW, n_chunk, _CHUNK))


_BT = 4096


def _dense_body(g_ref, s_ref, w_ref, pt_ref, gm_ref, bt_ref, o_ref):
    _, dim, bt = o_ref.shape
    g = g_ref[...]
    xt = lax.dot_general(w_ref[...], g, (((0,), (1,)), ((), ())),
                         preferred_element_type=jnp.float32)
    l = pl.program_id(0)
    s = s_ref[...].reshape(1, bt)
    ptid = lax.broadcast_in_dim(l * 3 + s, (64, bt), (0, 1))
    cls = lax.broadcasted_iota(jnp.int32, (64, bt), 0)
    oht = jnp.where(ptid == cls, 1.0, 0.0)
    xt = xt + lax.dot_general(pt_ref[...], oht, (((0,), (0,)), ((), ())),
                              preferred_element_type=jnp.float32)
    mean = jnp.sum(xt, axis=0, keepdims=True) * (1.0 / dim)
    xc = xt - mean
    var = jnp.sum(xc * xc, axis=0, keepdims=True) * (1.0 / dim)
    y = xc * lax.rsqrt(var + 1e-12)
    y = y * gm_ref[...] + bt_ref[...]
    o_ref[...] = y.reshape(1, dim, bt)


def _dense_body_acc(prev_ref, g_ref, s_ref, w_ref, pt_ref, gm_ref, bt_ref, o_ref):
    del prev_ref
    _dense_body(g_ref, s_ref, w_ref, pt_ref, gm_ref, bt_ref, o_ref)


def _tc_dense(g2, seg3, W2, PTa, gammaT, betaT, batch, L, b_off=0, prev=None):
    wd = W2.shape[0]
    dim = W2.shape[1]
    hb = seg3.shape[2]
    nbb = hb // _BT
    grid = (L, nbb)
    in_specs = [
        pl.BlockSpec((_BT, wd), lambda l, j: (l * nbb + j, 0)),
        pl.BlockSpec((1, 1, _BT), lambda l, j: (l, 0, j)),
        pl.BlockSpec((wd, dim), lambda l, j: (0, 0)),
        pl.BlockSpec((64, dim), lambda l, j: (0, 0)),
        pl.BlockSpec((dim, 1), lambda l, j: (0, 0)),
        pl.BlockSpec((dim, 1), lambda l, j: (0, 0)),
    ]
    args = (g2, seg3, W2, PTa, gammaT, betaT)
    body = _dense_body
    kwargs = {}
    if prev is not None:
        in_specs = [pl.BlockSpec(memory_space=pl.ANY)] + in_specs
        args = (prev,) + args
        body = _dense_body_acc
        kwargs["input_output_aliases"] = {0: 0}
    return pl.pallas_call(
        body,
        grid=grid,
        in_specs=in_specs,
        out_specs=pl.BlockSpec((1, dim, _BT), lambda l, j: (l, 0, j + b_off)),
        out_shape=jax.ShapeDtypeStruct((L, dim, batch), jnp.float32),
        **kwargs,
    )(*args)


def kernel(input_ids, segment_ids, word_emb, W2, pos_emb, type_emb, gamma, beta):
    batch, L = input_ids.shape
    dim = W2.shape[1]
    idsT = input_ids.astype(jnp.int32).T
    segT = segment_ids.astype(jnp.int32).T
    gammaT = gamma.reshape(-1, 1)
    betaT = beta.reshape(-1, 1)

    PTa = jnp.zeros((64, dim), jnp.float32)
    PTa = PTa.at[: 3 * L].set(
        (pos_emb[:, None, :] + type_emb[None, :, :]).reshape(3 * L, dim))

    nsplit = 4
    hb = batch // nsplit
    out = None
    for q in range(nsplit):
        ids_q = idsT[:, q * hb:(q + 1) * hb].reshape(-1)
        g_q = _sc_gather(word_emb, ids_q)
        seg3_q = segT[:, q * hb:(q + 1) * hb].reshape(L, 1, hb)
        out = _tc_dense(g_q, seg3_q, W2, PTa, gammaT, betaT, batch, L,
                        b_off=q * (hb // _BT), prev=out)
    return jnp.transpose(out, (2, 0, 1))
